# baseline (speedup 1.0000x reference)
import functools

import jax
import jax.numpy as jnp
from jax import lax
from jax.experimental import pallas as pl
from jax.experimental.pallas import tpu as pltpu
from jax.experimental.pallas import tpu_sc as plsc

_NUM_CORES = 2
_NUM_SUBCORES = 16
_REP = 128


def kernel(inputs, pos_embeddings):
    batch, seq, dims = inputs.shape
    row = seq * dims
    pos = pos_embeddings[:seq, :].reshape(row)

    b_per_core = batch // _NUM_CORES
    rep = _REP
    while b_per_core % (rep * _NUM_SUBCORES):
        rep //= 2
    n_dma = b_per_core // (rep * _NUM_SUBCORES)
    fill_per_sub = rep // _NUM_SUBCORES

    mesh = plsc.VectorSubcoreMesh(
        core_axis_name="c",
        subcore_axis_name="s",
        num_cores=_NUM_CORES,
        num_subcores=_NUM_SUBCORES,
    )

    @functools.partial(
        pl.kernel,
        out_type=jax.ShapeDtypeStruct((batch * row,), jnp.float32),
        mesh=mesh,
        scratch_types=[
            pltpu.VMEM_SHARED((rep * row,), jnp.float32),
            pltpu.SemaphoreType.DMA,
            pltpu.SemaphoreType.DMA,
        ],
    )
    def run(pos_hbm, out_hbm, buf, sem_in, sem_out):
        cid = lax.axis_index("c")
        sid = lax.axis_index("s")
        fills = [
            pltpu.async_copy(
                pos_hbm,
                buf.at[pl.ds((sid * fill_per_sub + r) * row, row)],
                sem_in,
            )
            for r in range(fill_per_sub)
        ]
        for cp in fills:
            cp.wait()
        plsc.subcore_barrier()
        chunk = rep * row
        base = cid * (b_per_core * row) + sid * (n_dma * chunk)
        outs = [
            pltpu.async_copy(buf, out_hbm.at[pl.ds(base + j * chunk, chunk)], sem_out)
            for j in range(n_dma)
        ]
        for cp in outs:
            cp.wait()

    return run(pos).reshape(batch, seq, dims)


# speedup vs baseline: 1.0557x; 1.0557x over previous
import functools

import jax
import jax.numpy as jnp
from jax import lax
from jax.experimental import pallas as pl
from jax.experimental.pallas import tpu as pltpu
from jax.experimental.pallas import tpu_sc as plsc

_NUM_CORES = 2
_NUM_SUBCORES = 16
_REP_T = 4
_REP_S = 96


def kernel(inputs, pos_embeddings):
    batch, seq, dims = inputs.shape
    row = seq * dims
    pos = pos_embeddings[:seq, :].reshape(row)

    nw = _NUM_CORES * _NUM_SUBCORES
    b_per_w = batch // nw
    half = b_per_w // 2
    n_t = half // _REP_T
    s_rep = min(_REP_S, half)
    n_s = half // s_rep
    fill_s = _REP_S // _NUM_SUBCORES

    mesh = plsc.VectorSubcoreMesh(
        core_axis_name="c",
        subcore_axis_name="s",
        num_cores=_NUM_CORES,
        num_subcores=_NUM_SUBCORES,
    )

    @functools.partial(
        pl.kernel,
        out_type=jax.ShapeDtypeStruct((batch * row,), jnp.float32),
        mesh=mesh,
        scratch_types=[
            pltpu.VMEM((_REP_T * row,), jnp.float32),
            pltpu.VMEM_SHARED((_REP_S * row,), jnp.float32),
            pltpu.SemaphoreType.DMA,
            pltpu.SemaphoreType.DMA,
            pltpu.SemaphoreType.DMA,
        ],
    )
    def run(pos_hbm, out_hbm, buf_t, buf_s, sem_in, sem_t, sem_s):
        cid = lax.axis_index("c")
        sid = lax.axis_index("s")
        fills = [
            pltpu.async_copy(pos_hbm, buf_t.at[pl.ds(r * row, row)], sem_in)
            for r in range(_REP_T)
        ] + [
            pltpu.async_copy(
                pos_hbm,
                buf_s.at[pl.ds((sid * fill_s + r) * row, row)],
                sem_in,
            )
            for r in range(fill_s)
        ]
        for cp in fills:
            cp.wait()
        plsc.subcore_barrier()

        base = (cid * _NUM_SUBCORES + sid) * (b_per_w * row)
        outs = [
            pltpu.async_copy(
                buf_t,
                out_hbm.at[pl.ds(base + j * (_REP_T * row), _REP_T * row)],
                sem_t,
            )
            for j in range(n_t)
        ] + [
            pltpu.async_copy(
                buf_s.at[pl.ds(0, s_rep * row)],
                out_hbm.at[pl.ds(base + (half + j * s_rep) * row, s_rep * row)],
                sem_s,
            )
            for j in range(n_s)
        ]
        for cp in outs:
            cp.wait()

    return run(pos).reshape(batch, seq, dims)


# speedup vs baseline: 1.0985x; 1.0405x over previous
import jax
import jax.numpy as jnp
from jax.experimental import pallas as pl
from jax.experimental.pallas import tpu as pltpu

_NOUT = 8
_BB = 32


def _tile_kernel(pos_ref, *out_refs):
    for o in out_refs:
        o[...] = jnp.broadcast_to(pos_ref[...][None, :, :], o.shape)


def kernel(inputs, pos_embeddings):
    batch, seq, dims = inputs.shape
    pos = pos_embeddings[:seq, :]
    sub = batch // _NOUT
    grid = (sub // _BB,)

    outs = pl.pallas_call(
        _tile_kernel,
        grid=grid,
        in_specs=[pl.BlockSpec((seq, dims), lambda i: (0, 0))],
        out_specs=[pl.BlockSpec((_BB, seq, dims), lambda i: (i, 0, 0))] * _NOUT,
        out_shape=[jax.ShapeDtypeStruct((sub, seq, dims), pos.dtype)] * _NOUT,
    )(pos)
    return jnp.concatenate(outs, axis=0)
